# baseline (speedup 1.0000x reference)
import functools

import jax
import jax.numpy as jnp
from jax import lax
from jax.experimental import pallas as pl
from jax.experimental.pallas import tpu as pltpu
from jax.experimental.pallas import tpu_sc as plsc

N = 10000
NPAD = 10240
C = 128
CH = 64
CW = 64
NG = 64
NLAYER = 3
BN = 1000
NB = N // BN
BE = 2000
F32 = jnp.float32


def _atttab_body(eets_ref, We_ref, aedge_ref, W_ref, asrc_ref, adst_ref,
                 t_ref, was_ref, wad_ref):
    v = jnp.dot(We_ref[...], aedge_ref[...], preferred_element_type=F32)
    t_ref[...] = 0.5 * jnp.dot(eets_ref[...], v, preferred_element_type=F32)
    was_ref[...] = jnp.dot(W_ref[...], asrc_ref[...],
                           preferred_element_type=F32)
    wad_ref[...] = jnp.dot(W_ref[...], adst_ref[...],
                           preferred_element_type=F32)


def _atttab(eets, We, aedge, W, asrc, adst):
    return pl.pallas_call(
        _atttab_body,
        out_shape=[
            jax.ShapeDtypeStruct((256, 1), F32),
            jax.ShapeDtypeStruct((128, 1), F32),
            jax.ShapeDtypeStruct((128, 1), F32),
        ],
    )(eets, We, aedge, W, asrc, adst)


def _k0x_body(nif_ref, nff_ref, nit_ref, Wf_ref, bf_ref, x_ref):
    nif = nif_ref[...]
    lanes = lax.broadcasted_iota(jnp.int32, (BN, 128), 1)
    acc = jnp.zeros((BN, 128), F32)
    for k in range(3):
        oh = (nif[:, k:k + 1] == lanes).astype(F32)
        acc += jnp.dot(oh, nit_ref[128 * k:128 * (k + 1), :],
                       preferred_element_type=F32)
    x_ref[...] = (acc * (1.0 / 3.0)
                  + jnp.dot(nff_ref[...], Wf_ref[...],
                            preferred_element_type=F32) + bf_ref[...])


def _k0x(nif, nff, nitp, Wf, bf2):
    return pl.pallas_call(
        _k0x_body,
        grid=(NB,),
        in_specs=[
            pl.BlockSpec((BN, 3), lambda i: (i, 0)),
            pl.BlockSpec((BN, 4), lambda i: (i, 0)),
            pl.BlockSpec((384, 128), lambda i: (0, 0)),
            pl.BlockSpec((4, 128), lambda i: (0, 0)),
            pl.BlockSpec((1, 128), lambda i: (0, 0)),
        ],
        out_specs=pl.BlockSpec((BN, 128), lambda i: (i, 0)),
        out_shape=jax.ShapeDtypeStruct((N, 128), F32),
    )(nif, nff, nitp, Wf, bf2)


def _kprep_body(x_ref, Wh_ref, was_ref, wad_ref, haug_ref, hs_ref, hd_ref):
    x = x_ref[...]
    haug_ref[...] = jnp.dot(x, Wh_ref[0], preferred_element_type=F32)
    hs_ref[...] = jnp.dot(x, was_ref[...], preferred_element_type=F32)
    hd_ref[...] = jnp.dot(x, wad_ref[...], preferred_element_type=F32)


def _kprep(x, Wsplit, was, wad):
    return pl.pallas_call(
        _kprep_body,
        grid=(2, NB),
        in_specs=[
            pl.BlockSpec((BN, 128), lambda p, i: (i, 0)),
            pl.BlockSpec((1, 128, CH), lambda p, i: (p, 0, 0)),
            pl.BlockSpec((128, 1), lambda p, i: (0, 0)),
            pl.BlockSpec((128, 1), lambda p, i: (0, 0)),
        ],
        out_specs=[
            pl.BlockSpec((BN, CW), lambda p, i: (p * NB + i, 0)),
            pl.BlockSpec((BN, 1), lambda p, i: (i, 0)),
            pl.BlockSpec((BN, 1), lambda p, i: (i, 0)),
        ],
        out_shape=[
            jax.ShapeDtypeStruct((2 * N, CW), F32),
            jax.ShapeDtypeStruct((N, 1), F32),
            jax.ShapeDtypeStruct((N, 1), F32),
        ],
    )(x, Wsplit, was, wad)


def _finalize(numA_ref, numB_ref, hA_ref, hB_ref, hsp_ref, hdp_ref, den_ref,
              ts_ref, bprev_ref, inv_e):
    c = jnp.sum(ts_ref[...]) * inv_e
    al = hsp_ref[...] + hdp_ref[...] + c
    exlo = jnp.exp(jnp.where(al >= 0, al, 0.2 * al))
    den = den_ref[...] + exlo
    xA = (numA_ref[...] + exlo * hA_ref[...]) / den + bprev_ref[:, 0:CH]
    xB = (numB_ref[...] + exlo * hB_ref[...]) / den + bprev_ref[:, CH:2 * CH]
    return xA, xB


def _kmid_body(numA_ref, numB_ref, hA_ref, hB_ref, hsp_ref, hdp_ref, den_ref,
               ts_ref, bprev_ref, x_ref, *, inv_e):
    xA, xB = _finalize(numA_ref, numB_ref, hA_ref, hB_ref, hsp_ref, hdp_ref,
                       den_ref, ts_ref, bprev_ref, inv_e)
    x_ref[:, 0:CH] = jnp.maximum(xA, 0.0)
    x_ref[:, CH:2 * CH] = jnp.maximum(xB, 0.0)


def _kmid(num, den2, csum, haug, hsp, hdp, bprev2, E):
    return pl.pallas_call(
        functools.partial(_kmid_body, inv_e=1.0 / E),
        grid=(NB,),
        in_specs=[
            pl.BlockSpec((BN, CW), lambda i: (i, 0)),
            pl.BlockSpec((BN, CW), lambda i: (i, 0)),
            pl.BlockSpec((BN, CW), lambda i: (i, 0)),
            pl.BlockSpec((BN, CW), lambda i: (NB + i, 0)),
            pl.BlockSpec((BN, 1), lambda i: (i, 0)),
            pl.BlockSpec((BN, 1), lambda i: (i, 0)),
            pl.BlockSpec((BN, 1), lambda i: (i, 0)),
            pl.BlockSpec((16, 16), lambda i: (0, 0)),
            pl.BlockSpec((1, 128), lambda i: (0, 0)),
        ],
        out_specs=pl.BlockSpec((BN, 128), lambda i: (i, 0)),
        out_shape=jax.ShapeDtypeStruct((N, 128), F32),
    )(num[0], num[1], haug, haug, hsp, hdp, den2, csum, bprev2)


def _kfin_body(numA_ref, numB_ref, hA_ref, hB_ref, hsp_ref, hdp_ref, den_ref,
               ts_ref, bprev_ref, bid_ref, out_ref, *, inv_e):
    i = pl.program_id(0)
    xA, xB = _finalize(numA_ref, numB_ref, hA_ref, hB_ref, hsp_ref, hdp_ref,
                       den_ref, ts_ref, bprev_ref, inv_e)
    bid = bid_ref[0]
    rows = lax.broadcasted_iota(jnp.int32, (NG, BN), 0)
    oh = (bid == rows).astype(F32)

    @pl.when(i == 0)
    def _():
        out_ref[...] = jnp.zeros_like(out_ref)

    out_ref[:, 0:CH] += jnp.dot(oh, xA, preferred_element_type=F32)
    out_ref[:, CH:2 * CH] += jnp.dot(oh, xB, preferred_element_type=F32)


def _kfin(num, den2, csum, haug, hsp, hdp, bprev2, bid3, E):
    return pl.pallas_call(
        functools.partial(_kfin_body, inv_e=1.0 / E),
        grid=(NB,),
        in_specs=[
            pl.BlockSpec((BN, CW), lambda i: (i, 0)),
            pl.BlockSpec((BN, CW), lambda i: (i, 0)),
            pl.BlockSpec((BN, CW), lambda i: (i, 0)),
            pl.BlockSpec((BN, CW), lambda i: (NB + i, 0)),
            pl.BlockSpec((BN, 1), lambda i: (i, 0)),
            pl.BlockSpec((BN, 1), lambda i: (i, 0)),
            pl.BlockSpec((BN, 1), lambda i: (i, 0)),
            pl.BlockSpec((16, 16), lambda i: (0, 0)),
            pl.BlockSpec((1, 128), lambda i: (0, 0)),
            pl.BlockSpec((1, 1, BN), lambda i: (i, 0, 0)),
        ],
        out_specs=pl.BlockSpec((NG, C), lambda i: (0, 0)),
        out_shape=jax.ShapeDtypeStruct((NG, C), F32),
    )(num[0], num[1], haug, haug, hsp, hdp, den2, csum, bprev2, bid3)


@functools.lru_cache(maxsize=None)
def _make_sc_edge(E):
    RTP = (E // 128 + 127) // 128 * 128
    CNT = RTP // 16
    mesh = plsc.VectorSubcoreMesh(core_axis_name="c", subcore_axis_name="s",
                                  num_cores=2, num_subcores=16)
    rows_per_sub = NPAD // 16

    @functools.partial(
        pl.kernel,
        out_type=[jax.ShapeDtypeStruct((2, NPAD, CW), F32),
                  jax.ShapeDtypeStruct((2, NPAD // 16, 16), F32),
                  jax.ShapeDtypeStruct((2, 16, 16), F32)],
        mesh=mesh,
        compiler_params=pltpu.CompilerParams(needs_layout_passes=False,
                                             use_tc_tiling_on_sc=False),
        scratch_types=[
            pltpu.VMEM_SHARED((NPAD, CW), F32),
            pltpu.VMEM_SHARED((NPAD // 16, 16), F32),
            pltpu.VMEM_SHARED((16, 16), F32),
            pltpu.VMEM((N,), F32),
            pltpu.VMEM((N,), F32),
            pltpu.VMEM((256,), F32),
            pltpu.VMEM((16,), F32),
            pltpu.VMEM((NPAD // 16, 16), F32),
            pltpu.VMEM((128,), jnp.int32),
            pltpu.VMEM((2, 4, 128), jnp.int32),
            pltpu.VMEM((2, 144), F32),
            pltpu.VMEM((2, 128, CW), F32),
            pltpu.SemaphoreType.DMA,
            pltpu.SemaphoreType.DMA,
            pltpu.SemaphoreType.DMA,
            pltpu.SemaphoreType.DMA,
        ],
    )
    def sc_edge(haug_hbm, idx4R, hs_hbm, hd_hbm, t_hbm,
                out_hbm, den_hbm, csum_hbm, num_sp, den_sp, csum_sp,
                hs_v, hd_v, t_v, tacc_v, den_v, iotab,
                i4, ex_v, buf, gsem0, gsem1, ssem0, ssem1):
        cid = lax.axis_index("c")
        sid = lax.axis_index("s")
        row_off = cid * N
        gsems = (gsem0, gsem1)
        ssems = (ssem0, ssem1)

        pltpu.sync_copy(hs_hbm, hs_v)
        pltpu.sync_copy(hd_hbm, hd_v)
        pltpu.sync_copy(t_hbm, t_v)

        def _zrow(k, carry):
            for j in range(CW // 16):
                buf[0, k, pl.ds(j * 16, 16)] = jnp.zeros((16,), F32)
            return carry

        lax.fori_loop(0, 128, _zrow, 0)
        tacc_v[...] = jnp.zeros((16,), F32)

        def _zden(k, carry):
            den_v[k, pl.ds(0, 16)] = jnp.zeros((16,), F32)
            return carry

        lax.fori_loop(0, NPAD // 16, _zden, 0)
        for i in range(8):
            iotab[pl.ds(16 * i, 16)] = lax.iota(jnp.int32, 16) + 16 * i
        base = sid * rows_per_sub
        for i in range(rows_per_sub // 128):
            pltpu.sync_copy(buf.at[0],
                            num_sp.at[pl.ds(base + i * 128, 128)])

        @pl.when(sid == 0)
        def _():
            pltpu.sync_copy(den_v, den_sp)

        plsc.subcore_barrier()

        lo = sid * CNT

        def _phase_a(s):
            for i in range(8):
                sv = i4[s, 0, pl.ds(i * 16, 16)]
                dv = i4[s, 1, pl.ds(i * 16, 16)]
                a0v = i4[s, 2, pl.ds(i * 16, 16)]
                a1v = i4[s, 3, pl.ds(i * 16, 16)]
                tg = (plsc.load_gather(t_v, [a0v])
                      + plsc.load_gather(t_v, [a1v + 128]))
                tacc_v[...] += tg
                g = (plsc.load_gather(hs_v, [sv])
                     + plsc.load_gather(hd_v, [dv]) + tg)
                g = jnp.where(g >= 0, g, 0.2 * g)
                exv = jnp.exp(g)
                ex_v[s, pl.ds(i * 16, 16)] = exv
                plsc.addupdate_scatter(den_v, [dv // 16, dv % 16], exv)
                i4[s, 0, pl.ds(i * 16, 16)] = sv + row_off

        def _mul(s):
            @functools.partial(plsc.parallel_loop, 0, 128, unroll=4)
            def _mrow(k):
                e = ex_v[s, pl.ds(k, 16)][0]
                for jj in range(CW // 16):
                    buf[s, k, pl.ds(jj * 16, 16)] = (
                        buf[s, k, pl.ds(jj * 16, 16)] * e)

        def _drain_scatter(s):
            pltpu.make_async_copy(buf.at[s], num_sp.at[i4.at[s, 1]],
                                  ssems[s]).wait()

        def _pair(p, carry):
            a = lo + 2 * p
            gd = [None, None]
            for s in (0, 1):
                @pl.when(p > 0)
                def _():
                    _drain_scatter(s)

                pltpu.sync_copy(idx4R.at[a + s], i4.at[s])
                _phase_a(s)
                gd[s] = pltpu.async_copy(haug_hbm.at[i4.at[s, 0]],
                                         buf.at[s], gsems[s])
            for s in (0, 1):
                gd[s].wait()
                _mul(s)
                pltpu.async_copy(buf.at[s], num_sp.at[i4.at[s, 1]],
                                 ssems[s], add=True)
            return carry

        lax.fori_loop(0, CNT // 2, _pair, 0)
        _drain_scatter(0)
        _drain_scatter(1)

        for j in range(NPAD // 16 // 128):
            pltpu.sync_copy(den_v.at[pl.ds(128 * j, 128)],
                            den_sp.at[iotab], add=True)
            for i in range(8):
                iotab[pl.ds(16 * i, 16)] = iotab[pl.ds(16 * i, 16)] + 128

        pltpu.sync_copy(tacc_v, csum_sp.at[sid])
        plsc.subcore_barrier()
        pltpu.sync_copy(num_sp.at[pl.ds(base, rows_per_sub)],
                        out_hbm.at[cid, pl.ds(base, rows_per_sub)])

        @pl.when(sid == 0)
        def _():
            pltpu.sync_copy(csum_sp, csum_hbm.at[cid])
            pltpu.sync_copy(den_sp, den_hbm.at[cid])

    return sc_edge


def _sc_edge_call(haug, idx4R, hs, hd, t, E):
    return _make_sc_edge(E)(haug, idx4R, hs, hd, t)


def _graph_forward(nif, nff, ea, ei, bid, nitp, Wf, bf2, gat, E):
    srcR = ei[0].reshape(E // 128, 128)
    dstR = ei[1].reshape(E // 128, 128)
    a0R = jnp.asarray(ea[:, 0]).reshape(E // 128, 128)
    a1R = jnp.asarray(ea[:, 1]).reshape(E // 128, 128)
    idx4R = jnp.stack([srcR, dstR, a0R, a1R], axis=1)
    RT = E // 128
    RTP = (RT + 127) // 128 * 128
    spread = jnp.arange((RTP - RT) * 128, dtype=jnp.int32) % 224
    pad = jnp.concatenate([
        spread.reshape(RTP - RT, 1, 128),
        (N + spread).reshape(RTP - RT, 1, 128),
        jnp.full((RTP - RT, 2, 128), 127, jnp.int32),
    ], axis=1)
    idx4R = jnp.concatenate([idx4R, pad], axis=0)
    bid3 = bid.reshape(NB, 1, BN)

    x = _k0x(nif, nff, nitp, Wf, bf2)
    for l in range(NLAYER):
        Wsplit, b, t, was, wad = gat[l]
        haug, hs, hd = _kprep(x, Wsplit, was, wad)
        num, den, csum = _sc_edge_call(haug, idx4R,
                                       hs.reshape(N), hd.reshape(N),
                                       t.reshape(256), E)
        num = num[:, :N, :]
        den2 = den[0].reshape(NPAD, 1)[:N]
        csum = csum[0]
        if l < NLAYER - 1:
            x = _kmid(num, den2, csum, haug, hs, hd, b, E)
        else:
            out = _kfin(num, den2, csum, haug, hs, hd, b, bid3, E)
    return out


def kernel(aa_nodes_int_feats, aa_nodes_float_feats, aa_edge_attrs, aa_edges, aa_batch_id, mol_nodes_int_feats, mol_nodes_float_feats, mol_edge_attrs, mol_edges, mol_batch_id, neighbor_nodes_int_feats, neighbor_nodes_float_feats, neighbor_edge_attrs, neighbor_edges, neighbor_batch_id, nit_0, nit_1, nit_2, eet_0, eet_1, Wf, bf, gat0_W, gat0_b, gat0_att_src, gat0_att_dst, gat0_att_edge, gat0_We, gat1_W, gat1_b, gat1_att_src, gat1_att_dst, gat1_att_edge, gat1_We, gat2_W, gat2_b, gat2_att_src, gat2_att_dst, gat2_att_edge, gat2_We):
    pad28 = lambda m: jnp.pad(m, ((0, 28), (0, 0)))
    nitp = jnp.concatenate([pad28(nit_0), pad28(nit_1), pad28(nit_2)], axis=0)
    eets = jnp.concatenate([pad28(eet_0), pad28(eet_1)], axis=0)
    bf2 = bf.reshape(1, 128)

    gparams = [
        (gat0_W, gat0_b, gat0_att_src, gat0_att_dst, gat0_att_edge, gat0_We),
        (gat1_W, gat1_b, gat1_att_src, gat1_att_dst, gat1_att_edge, gat1_We),
        (gat2_W, gat2_b, gat2_att_src, gat2_att_dst, gat2_att_edge, gat2_We),
    ]
    gat = []
    for (W, b, a_s, a_d, a_e, We) in gparams:
        t, was, wad = _atttab(eets, We, a_e.reshape(128, 1), W,
                              a_s.reshape(128, 1), a_d.reshape(128, 1))
        Wsplit = jnp.stack([W[:, 0:CH], W[:, CH:2 * CH]])
        gat.append((Wsplit, b.reshape(1, 128), t, was, wad))

    graphs = [
        (aa_nodes_int_feats, aa_nodes_float_feats, aa_edge_attrs, aa_edges,
         aa_batch_id, 320000),
        (mol_nodes_int_feats, mol_nodes_float_feats, mol_edge_attrs,
         mol_edges, mol_batch_id, 160000),
        (neighbor_nodes_int_feats, neighbor_nodes_float_feats,
         neighbor_edge_attrs, neighbor_edges, neighbor_batch_id, 160000),
    ]
    outs = [_graph_forward(nif, nff, ea, ei, bid, nitp, Wf, bf2, gat, E)
            for (nif, nff, ea, ei, bid, E) in graphs]
    return jnp.concatenate(outs, axis=0)


# speedup vs baseline: 1.0226x; 1.0226x over previous
import functools

import jax
import jax.numpy as jnp
from jax import lax
from jax.experimental import pallas as pl
from jax.experimental.pallas import tpu as pltpu
from jax.experimental.pallas import tpu_sc as plsc

N = 10000
NPAD = 10240
C = 128
CH = 64
CW = 80
NG = 64
NLAYER = 3
BN = 1000
NB = N // BN
BE = 2000
F32 = jnp.float32


def _atttab_body(eets_ref, We_ref, aedge_ref, W_ref, asrc_ref, adst_ref,
                 t_ref, was_ref, wad_ref):
    v = jnp.dot(We_ref[...], aedge_ref[...], preferred_element_type=F32)
    t_ref[...] = 0.5 * jnp.dot(eets_ref[...], v, preferred_element_type=F32)
    was_ref[...] = jnp.dot(W_ref[...], asrc_ref[...],
                           preferred_element_type=F32)
    wad_ref[...] = jnp.dot(W_ref[...], adst_ref[...],
                           preferred_element_type=F32)


def _atttab(eets, We, aedge, W, asrc, adst):
    return pl.pallas_call(
        _atttab_body,
        out_shape=[
            jax.ShapeDtypeStruct((256, 1), F32),
            jax.ShapeDtypeStruct((128, 1), F32),
            jax.ShapeDtypeStruct((128, 1), F32),
        ],
    )(eets, We, aedge, W, asrc, adst)


def _k0x_body(nif_ref, nff_ref, nit_ref, Wf_ref, bf_ref, x_ref):
    nif = nif_ref[...]
    lanes = lax.broadcasted_iota(jnp.int32, (BN, 128), 1)
    acc = jnp.zeros((BN, 128), F32)
    for k in range(3):
        oh = (nif[:, k:k + 1] == lanes).astype(F32)
        acc += jnp.dot(oh, nit_ref[128 * k:128 * (k + 1), :],
                       preferred_element_type=F32)
    x_ref[...] = (acc * (1.0 / 3.0)
                  + jnp.dot(nff_ref[...], Wf_ref[...],
                            preferred_element_type=F32) + bf_ref[...])


def _k0x(nif, nff, nitp, Wf, bf2):
    return pl.pallas_call(
        _k0x_body,
        grid=(NB,),
        in_specs=[
            pl.BlockSpec((BN, 3), lambda i: (i, 0)),
            pl.BlockSpec((BN, 4), lambda i: (i, 0)),
            pl.BlockSpec((384, 128), lambda i: (0, 0)),
            pl.BlockSpec((4, 128), lambda i: (0, 0)),
            pl.BlockSpec((1, 128), lambda i: (0, 0)),
        ],
        out_specs=pl.BlockSpec((BN, 128), lambda i: (i, 0)),
        out_shape=jax.ShapeDtypeStruct((N, 128), F32),
    )(nif, nff, nitp, Wf, bf2)


def _kprep_body(x_ref, Wh_ref, was_ref, wad_ref, haug_ref, hs_ref, hd_ref):
    x = x_ref[...]
    haug_ref[:, 0:CH] = jnp.dot(x, Wh_ref[0], preferred_element_type=F32)
    ex16 = (lax.broadcasted_iota(jnp.int32, (BN, 16), 1) == 0).astype(F32)
    haug_ref[:, CH:CW] = ex16
    hs_ref[...] = jnp.dot(x, was_ref[...], preferred_element_type=F32)
    hd_ref[...] = jnp.dot(x, wad_ref[...], preferred_element_type=F32)


def _kprep(x, Wsplit, was, wad):
    return pl.pallas_call(
        _kprep_body,
        grid=(2, NB),
        in_specs=[
            pl.BlockSpec((BN, 128), lambda p, i: (i, 0)),
            pl.BlockSpec((1, 128, CH), lambda p, i: (p, 0, 0)),
            pl.BlockSpec((128, 1), lambda p, i: (0, 0)),
            pl.BlockSpec((128, 1), lambda p, i: (0, 0)),
        ],
        out_specs=[
            pl.BlockSpec((BN, CW), lambda p, i: (p * NB + i, 0)),
            pl.BlockSpec((BN, 1), lambda p, i: (i, 0)),
            pl.BlockSpec((BN, 1), lambda p, i: (i, 0)),
        ],
        out_shape=[
            jax.ShapeDtypeStruct((2 * N, CW), F32),
            jax.ShapeDtypeStruct((N, 1), F32),
            jax.ShapeDtypeStruct((N, 1), F32),
        ],
    )(x, Wsplit, was, wad)


def _finalize(numA_ref, numB_ref, hA_ref, hB_ref, hsp_ref, hdp_ref, ts_ref,
              bprev_ref, inv_e):
    c = jnp.sum(ts_ref[...]) * inv_e
    al = hsp_ref[...] + hdp_ref[...] + c
    exlo = jnp.exp(jnp.where(al >= 0, al, 0.2 * al))
    den = numA_ref[:, CH:CH + 1] + exlo
    xA = ((numA_ref[:, 0:CH] + exlo * hA_ref[:, 0:CH]) / den
          + bprev_ref[:, 0:CH])
    xB = ((numB_ref[:, 0:CH] + exlo * hB_ref[:, 0:CH]) / den
          + bprev_ref[:, CH:2 * CH])
    return xA, xB


def _kmid_body(numA_ref, numB_ref, hA_ref, hB_ref, hsp_ref, hdp_ref, ts_ref,
               bprev_ref, x_ref, *, inv_e):
    xA, xB = _finalize(numA_ref, numB_ref, hA_ref, hB_ref, hsp_ref, hdp_ref,
                       ts_ref, bprev_ref, inv_e)
    x_ref[:, 0:CH] = jnp.maximum(xA, 0.0)
    x_ref[:, CH:2 * CH] = jnp.maximum(xB, 0.0)


def _kmid(num, csum, haug, hsp, hdp, bprev2, E):
    return pl.pallas_call(
        functools.partial(_kmid_body, inv_e=1.0 / E),
        grid=(NB,),
        in_specs=[
            pl.BlockSpec((BN, CW), lambda i: (i, 0)),
            pl.BlockSpec((BN, CW), lambda i: (i, 0)),
            pl.BlockSpec((BN, CW), lambda i: (i, 0)),
            pl.BlockSpec((BN, CW), lambda i: (NB + i, 0)),
            pl.BlockSpec((BN, 1), lambda i: (i, 0)),
            pl.BlockSpec((BN, 1), lambda i: (i, 0)),
            pl.BlockSpec((16, 16), lambda i: (0, 0)),
            pl.BlockSpec((1, 128), lambda i: (0, 0)),
        ],
        out_specs=pl.BlockSpec((BN, 128), lambda i: (i, 0)),
        out_shape=jax.ShapeDtypeStruct((N, 128), F32),
    )(num[0], num[1], haug, haug, hsp, hdp, csum, bprev2)


def _kfin_body(numA_ref, numB_ref, hA_ref, hB_ref, hsp_ref, hdp_ref, ts_ref,
               bprev_ref, bid_ref, out_ref, *, inv_e):
    i = pl.program_id(0)
    xA, xB = _finalize(numA_ref, numB_ref, hA_ref, hB_ref, hsp_ref, hdp_ref,
                       ts_ref, bprev_ref, inv_e)
    bid = bid_ref[0]
    rows = lax.broadcasted_iota(jnp.int32, (NG, BN), 0)
    oh = (bid == rows).astype(F32)

    @pl.when(i == 0)
    def _():
        out_ref[...] = jnp.zeros_like(out_ref)

    out_ref[:, 0:CH] += jnp.dot(oh, xA, preferred_element_type=F32)
    out_ref[:, CH:2 * CH] += jnp.dot(oh, xB, preferred_element_type=F32)


def _kfin(num, csum, haug, hsp, hdp, bprev2, bid3, E):
    return pl.pallas_call(
        functools.partial(_kfin_body, inv_e=1.0 / E),
        grid=(NB,),
        in_specs=[
            pl.BlockSpec((BN, CW), lambda i: (i, 0)),
            pl.BlockSpec((BN, CW), lambda i: (i, 0)),
            pl.BlockSpec((BN, CW), lambda i: (i, 0)),
            pl.BlockSpec((BN, CW), lambda i: (NB + i, 0)),
            pl.BlockSpec((BN, 1), lambda i: (i, 0)),
            pl.BlockSpec((BN, 1), lambda i: (i, 0)),
            pl.BlockSpec((16, 16), lambda i: (0, 0)),
            pl.BlockSpec((1, 128), lambda i: (0, 0)),
            pl.BlockSpec((1, 1, BN), lambda i: (i, 0, 0)),
        ],
        out_specs=pl.BlockSpec((NG, C), lambda i: (0, 0)),
        out_shape=jax.ShapeDtypeStruct((NG, C), F32),
    )(num[0], num[1], haug, haug, hsp, hdp, csum, bprev2, bid3)


@functools.lru_cache(maxsize=None)
def _make_sc_edge(E):
    RT = E // 128
    mesh = plsc.VectorSubcoreMesh(core_axis_name="c", subcore_axis_name="s",
                                  num_cores=2, num_subcores=16)
    rows_per_sub = NPAD // 16

    @functools.partial(
        pl.kernel,
        out_type=[jax.ShapeDtypeStruct((2, NPAD, CW), F32),
                  jax.ShapeDtypeStruct((2, 16, 16), F32)],
        mesh=mesh,
        compiler_params=pltpu.CompilerParams(needs_layout_passes=False,
                                             use_tc_tiling_on_sc=False),
        scratch_types=[
            pltpu.VMEM_SHARED((NPAD, CW), F32),
            pltpu.VMEM_SHARED((16, 16), F32),
            pltpu.VMEM((N,), F32),
            pltpu.VMEM((N,), F32),
            pltpu.VMEM((256,), F32),
            pltpu.VMEM((16,), F32),
            pltpu.VMEM((2, 4, 128), jnp.int32),
            pltpu.VMEM((2, 144), F32),
            pltpu.VMEM((2, 128, CW), F32),
            pltpu.SemaphoreType.DMA,
            pltpu.SemaphoreType.DMA,
            pltpu.SemaphoreType.DMA,
            pltpu.SemaphoreType.DMA,
        ],
    )
    def sc_edge(haug_hbm, idx4R, hs_hbm, hd_hbm, t_hbm,
                out_hbm, csum_hbm, num_sp, csum_sp,
                hs_v, hd_v, t_v, tacc_v,
                i4, ex_v, buf, gsem0, gsem1, ssem0, ssem1):
        cid = lax.axis_index("c")
        sid = lax.axis_index("s")
        row_off = cid * N
        gsems = (gsem0, gsem1)
        ssems = (ssem0, ssem1)

        pltpu.sync_copy(hs_hbm, hs_v)
        pltpu.sync_copy(hd_hbm, hd_v)
        pltpu.sync_copy(t_hbm, t_v)

        def _zrow(k, carry):
            for j in range(CW // 16):
                buf[0, k, pl.ds(j * 16, 16)] = jnp.zeros((16,), F32)
            return carry

        lax.fori_loop(0, 128, _zrow, 0)
        tacc_v[...] = jnp.zeros((16,), F32)
        base = sid * rows_per_sub
        for i in range(rows_per_sub // 128):
            pltpu.sync_copy(buf.at[0],
                            num_sp.at[pl.ds(base + i * 128, 128)])
        plsc.subcore_barrier()

        lo = sid * RT // 16
        hi = (sid + 1) * RT // 16
        cnt = hi - lo
        npairs = cnt // 2

        def _phase_a(s):
            for i in range(8):
                sv = i4[s, 0, pl.ds(i * 16, 16)]
                dv = i4[s, 1, pl.ds(i * 16, 16)]
                a0v = i4[s, 2, pl.ds(i * 16, 16)]
                a1v = i4[s, 3, pl.ds(i * 16, 16)]
                tg = (plsc.load_gather(t_v, [a0v])
                      + plsc.load_gather(t_v, [a1v + 128]))
                tacc_v[...] += tg
                g = (plsc.load_gather(hs_v, [sv])
                     + plsc.load_gather(hd_v, [dv]) + tg)
                g = jnp.where(g >= 0, g, 0.2 * g)
                ex_v[s, pl.ds(i * 16, 16)] = jnp.exp(g)
                i4[s, 0, pl.ds(i * 16, 16)] = sv + row_off

        def _mul(s):
            @functools.partial(plsc.parallel_loop, 0, 128, unroll=4)
            def _mrow(k):
                e = ex_v[s, pl.ds(k, 16)][0]
                for jj in range(CW // 16):
                    buf[s, k, pl.ds(jj * 16, 16)] = (
                        buf[s, k, pl.ds(jj * 16, 16)] * e)

        def _drain_scatter(s):
            pltpu.make_async_copy(buf.at[s], num_sp.at[i4.at[s, 1]],
                                  ssems[s]).wait()

        def _pair(p, carry):
            a = lo + 2 * p
            gd = [None, None]
            for s in (0, 1):
                @pl.when(p > 0)
                def _():
                    _drain_scatter(s)

                pltpu.sync_copy(idx4R.at[a + s], i4.at[s])
                _phase_a(s)
                gd[s] = pltpu.async_copy(haug_hbm.at[i4.at[s, 0]],
                                         buf.at[s], gsems[s])
            for s in (0, 1):
                gd[s].wait()
                _mul(s)
                pltpu.async_copy(buf.at[s], num_sp.at[i4.at[s, 1]],
                                 ssems[s], add=True)
            return carry

        lax.fori_loop(0, npairs, _pair, 0)

        @pl.when(npairs > 0)
        def _():
            _drain_scatter(0)
            _drain_scatter(1)

        @pl.when(cnt % 2 == 1)
        def _():
            pltpu.sync_copy(idx4R.at[hi - 1], i4.at[0])
            _phase_a(0)
            pltpu.async_copy(haug_hbm.at[i4.at[0, 0]], buf.at[0],
                             gsem0).wait()
            _mul(0)
            pltpu.sync_copy(buf.at[0], num_sp.at[i4.at[0, 1]], add=True)

        pltpu.sync_copy(tacc_v, csum_sp.at[sid])
        plsc.subcore_barrier()
        pltpu.sync_copy(num_sp.at[pl.ds(base, rows_per_sub)],
                        out_hbm.at[cid, pl.ds(base, rows_per_sub)])

        @pl.when(sid == 0)
        def _():
            pltpu.sync_copy(csum_sp, csum_hbm.at[cid])

    return sc_edge


def _sc_edge_call(haug, idx4R, hs, hd, t, E):
    return _make_sc_edge(E)(haug, idx4R, hs, hd, t)


def _graph_forward(nif, nff, ea, ei, bid, nitp, Wf, bf2, gat, E):
    srcR = ei[0].reshape(E // 128, 128)
    dstR = ei[1].reshape(E // 128, 128)
    a0R = jnp.asarray(ea[:, 0]).reshape(E // 128, 128)
    a1R = jnp.asarray(ea[:, 1]).reshape(E // 128, 128)
    idx4R = jnp.stack([srcR, dstR, a0R, a1R], axis=1)
    bid3 = bid.reshape(NB, 1, BN)

    x = _k0x(nif, nff, nitp, Wf, bf2)
    for l in range(NLAYER):
        Wsplit, b, t, was, wad = gat[l]
        haug, hs, hd = _kprep(x, Wsplit, was, wad)
        num, csum = _sc_edge_call(haug, idx4R,
                                  hs.reshape(N), hd.reshape(N),
                                  t.reshape(256), E)
        num = num[:, :N, :]
        csum = csum[0]
        if l < NLAYER - 1:
            x = _kmid(num, csum, haug, hs, hd, b, E)
        else:
            out = _kfin(num, csum, haug, hs, hd, b, bid3, E)
    return out


def kernel(aa_nodes_int_feats, aa_nodes_float_feats, aa_edge_attrs, aa_edges, aa_batch_id, mol_nodes_int_feats, mol_nodes_float_feats, mol_edge_attrs, mol_edges, mol_batch_id, neighbor_nodes_int_feats, neighbor_nodes_float_feats, neighbor_edge_attrs, neighbor_edges, neighbor_batch_id, nit_0, nit_1, nit_2, eet_0, eet_1, Wf, bf, gat0_W, gat0_b, gat0_att_src, gat0_att_dst, gat0_att_edge, gat0_We, gat1_W, gat1_b, gat1_att_src, gat1_att_dst, gat1_att_edge, gat1_We, gat2_W, gat2_b, gat2_att_src, gat2_att_dst, gat2_att_edge, gat2_We):
    pad28 = lambda m: jnp.pad(m, ((0, 28), (0, 0)))
    nitp = jnp.concatenate([pad28(nit_0), pad28(nit_1), pad28(nit_2)], axis=0)
    eets = jnp.concatenate([pad28(eet_0), pad28(eet_1)], axis=0)
    bf2 = bf.reshape(1, 128)

    gparams = [
        (gat0_W, gat0_b, gat0_att_src, gat0_att_dst, gat0_att_edge, gat0_We),
        (gat1_W, gat1_b, gat1_att_src, gat1_att_dst, gat1_att_edge, gat1_We),
        (gat2_W, gat2_b, gat2_att_src, gat2_att_dst, gat2_att_edge, gat2_We),
    ]
    gat = []
    for (W, b, a_s, a_d, a_e, We) in gparams:
        t, was, wad = _atttab(eets, We, a_e.reshape(128, 1), W,
                              a_s.reshape(128, 1), a_d.reshape(128, 1))
        Wsplit = jnp.stack([W[:, 0:CH], W[:, CH:2 * CH]])
        gat.append((Wsplit, b.reshape(1, 128), t, was, wad))

    graphs = [
        (aa_nodes_int_feats, aa_nodes_float_feats, aa_edge_attrs, aa_edges,
         aa_batch_id, 320000),
        (mol_nodes_int_feats, mol_nodes_float_feats, mol_edge_attrs,
         mol_edges, mol_batch_id, 160000),
        (neighbor_nodes_int_feats, neighbor_nodes_float_feats,
         neighbor_edge_attrs, neighbor_edges, neighbor_batch_id, 160000),
    ]
    outs = [_graph_forward(nif, nff, ea, ei, bid, nitp, Wf, bf2, gat, E)
            for (nif, nff, ea, ei, bid, E) in graphs]
    return jnp.concatenate(outs, axis=0)


# speedup vs baseline: 1.0232x; 1.0005x over previous
import functools

import jax
import jax.numpy as jnp
from jax import lax
from jax.experimental import pallas as pl
from jax.experimental.pallas import tpu as pltpu
from jax.experimental.pallas import tpu_sc as plsc

N = 10000
NPAD = 10240
C = 128
CH = 64
CW = 80
NG = 64
NLAYER = 3
BN = 1000
NB = N // BN
F32 = jnp.float32


def _atttab_body(eets_ref, We_ref, aedge_ref, W_ref, asrc_ref, adst_ref,
                 t_ref, was_ref, wad_ref):
    v = jnp.dot(We_ref[...], aedge_ref[...], preferred_element_type=F32)
    t_ref[...] = 0.5 * jnp.dot(eets_ref[...], v, preferred_element_type=F32)
    was_ref[...] = jnp.dot(W_ref[...], asrc_ref[...],
                           preferred_element_type=F32)
    wad_ref[...] = jnp.dot(W_ref[...], adst_ref[...],
                           preferred_element_type=F32)


def _atttab(eets, We, aedge, W, asrc, adst):
    return pl.pallas_call(
        _atttab_body,
        out_shape=[
            jax.ShapeDtypeStruct((256, 1), F32),
            jax.ShapeDtypeStruct((128, 1), F32),
            jax.ShapeDtypeStruct((128, 1), F32),
        ],
    )(eets, We, aedge, W, asrc, adst)


def _k0x_body(nif_ref, nff_ref, nit_ref, Wf_ref, bf_ref, x_ref):
    nif = nif_ref[...]
    lanes = lax.broadcasted_iota(jnp.int32, (BN, 128), 1)
    acc = jnp.zeros((BN, 128), F32)
    for k in range(3):
        oh = (nif[:, k:k + 1] == lanes).astype(F32)
        acc += jnp.dot(oh, nit_ref[128 * k:128 * (k + 1), :],
                       preferred_element_type=F32)
    x_ref[...] = (acc * (1.0 / 3.0)
                  + jnp.dot(nff_ref[...], Wf_ref[...],
                            preferred_element_type=F32) + bf_ref[...])


def _k0x(nif, nff, nitp, Wf, bf2):
    return pl.pallas_call(
        _k0x_body,
        grid=(NB,),
        in_specs=[
            pl.BlockSpec((BN, 3), lambda i: (i, 0)),
            pl.BlockSpec((BN, 4), lambda i: (i, 0)),
            pl.BlockSpec((384, 128), lambda i: (0, 0)),
            pl.BlockSpec((4, 128), lambda i: (0, 0)),
            pl.BlockSpec((1, 128), lambda i: (0, 0)),
        ],
        out_specs=pl.BlockSpec((BN, 128), lambda i: (i, 0)),
        out_shape=jax.ShapeDtypeStruct((N, 128), F32),
    )(nif, nff, nitp, Wf, bf2)


def _kprep_body(x_ref, Wh_ref, was_ref, wad_ref, haug_ref, hs_ref, hd_ref):
    x = x_ref[...]
    haug_ref[:, 0:CH] = jnp.dot(x, Wh_ref[0], preferred_element_type=F32)
    ex16 = (lax.broadcasted_iota(jnp.int32, (BN, 16), 1) == 0).astype(F32)
    haug_ref[:, CH:CW] = ex16
    hs_ref[...] = jnp.dot(x, was_ref[...], preferred_element_type=F32)
    hd_ref[...] = jnp.dot(x, wad_ref[...], preferred_element_type=F32)


def _kprep(x, Wsplit, was, wad):
    return pl.pallas_call(
        _kprep_body,
        grid=(2, NB),
        in_specs=[
            pl.BlockSpec((BN, 128), lambda p, i: (i, 0)),
            pl.BlockSpec((1, 128, CH), lambda p, i: (p, 0, 0)),
            pl.BlockSpec((128, 1), lambda p, i: (0, 0)),
            pl.BlockSpec((128, 1), lambda p, i: (0, 0)),
        ],
        out_specs=[
            pl.BlockSpec((BN, CW), lambda p, i: (p * NB + i, 0)),
            pl.BlockSpec((BN, 1), lambda p, i: (i, 0)),
            pl.BlockSpec((BN, 1), lambda p, i: (i, 0)),
        ],
        out_shape=[
            jax.ShapeDtypeStruct((2 * N, CW), F32),
            jax.ShapeDtypeStruct((N, 1), F32),
            jax.ShapeDtypeStruct((N, 1), F32),
        ],
    )(x, Wsplit, was, wad)


def _finalize(numA_ref, numB_ref, hA_ref, hB_ref, hsp_ref, hdp_ref, ts_ref,
              bprev_ref, inv_e):
    c = jnp.sum(ts_ref[...]) * inv_e
    al = hsp_ref[...] + hdp_ref[...] + c
    exlo = jnp.exp(jnp.where(al >= 0, al, 0.2 * al))
    den = numA_ref[:, CH:CH + 1] + exlo
    xA = ((numA_ref[:, 0:CH] + exlo * hA_ref[:, 0:CH]) / den
          + bprev_ref[:, 0:CH])
    xB = ((numB_ref[:, 0:CH] + exlo * hB_ref[:, 0:CH]) / den
          + bprev_ref[:, CH:2 * CH])
    return xA, xB


def _kmid_body(numA_ref, numB_ref, hA_ref, hB_ref, hsp_ref, hdp_ref, ts_ref,
               bprev_ref, x_ref, *, inv_e):
    xA, xB = _finalize(numA_ref, numB_ref, hA_ref, hB_ref, hsp_ref, hdp_ref,
                       ts_ref, bprev_ref, inv_e)
    x_ref[:, 0:CH] = jnp.maximum(xA, 0.0)
    x_ref[:, CH:2 * CH] = jnp.maximum(xB, 0.0)


def _kmid(num, csum, haug, hsp, hdp, bprev2, E):
    return pl.pallas_call(
        functools.partial(_kmid_body, inv_e=1.0 / E),
        grid=(NB,),
        in_specs=[
            pl.BlockSpec((BN, CW), lambda i: (i, 0)),
            pl.BlockSpec((BN, CW), lambda i: (i, 0)),
            pl.BlockSpec((BN, CW), lambda i: (i, 0)),
            pl.BlockSpec((BN, CW), lambda i: (NB + i, 0)),
            pl.BlockSpec((BN, 1), lambda i: (i, 0)),
            pl.BlockSpec((BN, 1), lambda i: (i, 0)),
            pl.BlockSpec((16, 16), lambda i: (0, 0)),
            pl.BlockSpec((1, 128), lambda i: (0, 0)),
        ],
        out_specs=pl.BlockSpec((BN, 128), lambda i: (i, 0)),
        out_shape=jax.ShapeDtypeStruct((N, 128), F32),
    )(num[0], num[1], haug, haug, hsp, hdp, csum, bprev2)


def _kfin_body(numA_ref, numB_ref, hA_ref, hB_ref, hsp_ref, hdp_ref, ts_ref,
               bprev_ref, bid_ref, out_ref, *, inv_e):
    i = pl.program_id(0)
    xA, xB = _finalize(numA_ref, numB_ref, hA_ref, hB_ref, hsp_ref, hdp_ref,
                       ts_ref, bprev_ref, inv_e)
    bid = bid_ref[0]
    rows = lax.broadcasted_iota(jnp.int32, (NG, BN), 0)
    oh = (bid == rows).astype(F32)

    @pl.when(i == 0)
    def _():
        out_ref[...] = jnp.zeros_like(out_ref)

    out_ref[:, 0:CH] += jnp.dot(oh, xA, preferred_element_type=F32)
    out_ref[:, CH:2 * CH] += jnp.dot(oh, xB, preferred_element_type=F32)


def _kfin(num, csum, haug, hsp, hdp, bprev2, bid3, E):
    return pl.pallas_call(
        functools.partial(_kfin_body, inv_e=1.0 / E),
        grid=(NB,),
        in_specs=[
            pl.BlockSpec((BN, CW), lambda i: (i, 0)),
            pl.BlockSpec((BN, CW), lambda i: (i, 0)),
            pl.BlockSpec((BN, CW), lambda i: (i, 0)),
            pl.BlockSpec((BN, CW), lambda i: (NB + i, 0)),
            pl.BlockSpec((BN, 1), lambda i: (i, 0)),
            pl.BlockSpec((BN, 1), lambda i: (i, 0)),
            pl.BlockSpec((16, 16), lambda i: (0, 0)),
            pl.BlockSpec((1, 128), lambda i: (0, 0)),
            pl.BlockSpec((1, 1, BN), lambda i: (i, 0, 0)),
        ],
        out_specs=pl.BlockSpec((NG, C), lambda i: (0, 0)),
        out_shape=jax.ShapeDtypeStruct((NG, C), F32),
    )(num[0], num[1], haug, haug, hsp, hdp, csum, bprev2, bid3)


@functools.lru_cache(maxsize=None)
def _make_sc_edge(E):
    RT = E // 128
    mesh = plsc.VectorSubcoreMesh(core_axis_name="c", subcore_axis_name="s",
                                  num_cores=2, num_subcores=16)
    rows_per_sub = NPAD // 16

    @functools.partial(
        pl.kernel,
        out_type=[jax.ShapeDtypeStruct((2, NPAD, CW), F32),
                  jax.ShapeDtypeStruct((2, 16, 16), F32)],
        mesh=mesh,
        compiler_params=pltpu.CompilerParams(needs_layout_passes=False,
                                             use_tc_tiling_on_sc=False),
        scratch_types=[
            pltpu.VMEM_SHARED((NPAD, CW), F32),
            pltpu.VMEM_SHARED((16, 16), F32),
            pltpu.VMEM((N,), F32),
            pltpu.VMEM((N,), F32),
            pltpu.VMEM((256,), F32),
            pltpu.VMEM((16,), F32),
            pltpu.VMEM((2, 4, 128), jnp.int32),
            pltpu.VMEM((2, 144), F32),
            pltpu.VMEM((2, 128, CW), F32),
            pltpu.SemaphoreType.DMA,
            pltpu.SemaphoreType.DMA,
            pltpu.SemaphoreType.DMA,
            pltpu.SemaphoreType.DMA,
        ],
    )
    def sc_edge(haug_hbm, idx4R, hs_hbm, hd_hbm, t_hbm,
                out_hbm, csum_hbm, num_sp, csum_sp,
                hs_v, hd_v, t_v, tacc_v,
                i4, ex_v, buf, gsem0, gsem1, ssem0, ssem1):
        cid = lax.axis_index("c")
        sid = lax.axis_index("s")
        row_off = cid * N
        gsems = (gsem0, gsem1)
        ssems = (ssem0, ssem1)

        pltpu.sync_copy(hs_hbm, hs_v)
        pltpu.sync_copy(hd_hbm, hd_v)
        pltpu.sync_copy(t_hbm, t_v)

        def _zrow(k, carry):
            for j in range(CW // 16):
                buf[0, k, pl.ds(j * 16, 16)] = jnp.zeros((16,), F32)
            return carry

        lax.fori_loop(0, 128, _zrow, 0)
        tacc_v[...] = jnp.zeros((16,), F32)
        base = sid * rows_per_sub
        for i in range(rows_per_sub // 128):
            pltpu.sync_copy(buf.at[0],
                            num_sp.at[pl.ds(base + i * 128, 128)])
        plsc.subcore_barrier()

        lo = sid * RT // 16
        hi = (sid + 1) * RT // 16
        cnt = hi - lo
        npairs = cnt // 2

        def _phase_a(s):
            for i in range(8):
                sv = i4[s, 0, pl.ds(i * 16, 16)]
                dv = i4[s, 1, pl.ds(i * 16, 16)]
                a0v = i4[s, 2, pl.ds(i * 16, 16)]
                a1v = i4[s, 3, pl.ds(i * 16, 16)]
                tg = (plsc.load_gather(t_v, [a0v])
                      + plsc.load_gather(t_v, [a1v + 128]))
                tacc_v[...] += tg
                g = (plsc.load_gather(hs_v, [sv])
                     + plsc.load_gather(hd_v, [dv]) + tg)
                g = jnp.where(g >= 0, g, 0.2 * g)
                ex_v[s, pl.ds(i * 16, 16)] = jnp.exp(g)
                i4[s, 0, pl.ds(i * 16, 16)] = sv + row_off

        def _mul(s):
            @functools.partial(plsc.parallel_loop, 0, 128, unroll=8)
            def _mrow(k):
                e = ex_v[s, pl.ds(k, 16)][0]
                for jj in range(CW // 16):
                    buf[s, k, pl.ds(jj * 16, 16)] = (
                        buf[s, k, pl.ds(jj * 16, 16)] * e)

        def _drain_scatter(s):
            pltpu.make_async_copy(buf.at[s], num_sp.at[i4.at[s, 1]],
                                  ssems[s]).wait()

        def _pair(p, carry):
            a = lo + 2 * p
            gd = [None, None]
            for s in (0, 1):
                @pl.when(p > 0)
                def _():
                    _drain_scatter(s)

                pltpu.sync_copy(idx4R.at[a + s], i4.at[s])
                _phase_a(s)
                gd[s] = pltpu.async_copy(haug_hbm.at[i4.at[s, 0]],
                                         buf.at[s], gsems[s])
            for s in (0, 1):
                gd[s].wait()
                _mul(s)
                pltpu.async_copy(buf.at[s], num_sp.at[i4.at[s, 1]],
                                 ssems[s], add=True)
            return carry

        lax.fori_loop(0, npairs, _pair, 0)

        @pl.when(npairs > 0)
        def _():
            _drain_scatter(0)
            _drain_scatter(1)

        @pl.when(cnt % 2 == 1)
        def _():
            pltpu.sync_copy(idx4R.at[hi - 1], i4.at[0])
            _phase_a(0)
            pltpu.async_copy(haug_hbm.at[i4.at[0, 0]], buf.at[0],
                             gsem0).wait()
            _mul(0)
            pltpu.sync_copy(buf.at[0], num_sp.at[i4.at[0, 1]], add=True)

        pltpu.sync_copy(tacc_v, csum_sp.at[sid])
        plsc.subcore_barrier()
        pltpu.sync_copy(num_sp.at[pl.ds(base, rows_per_sub)],
                        out_hbm.at[cid, pl.ds(base, rows_per_sub)])

        @pl.when(sid == 0)
        def _():
            pltpu.sync_copy(csum_sp, csum_hbm.at[cid])

    return sc_edge


def _sc_edge_call(haug, idx4R, hs, hd, t, E):
    return _make_sc_edge(E)(haug, idx4R, hs, hd, t)


def _graph_forward(nif, nff, ea, ei, bid, nitp, Wf, bf2, gat, E):
    srcR = ei[0].reshape(E // 128, 128)
    dstR = ei[1].reshape(E // 128, 128)
    a0R = jnp.asarray(ea[:, 0]).reshape(E // 128, 128)
    a1R = jnp.asarray(ea[:, 1]).reshape(E // 128, 128)
    idx4R = jnp.stack([srcR, dstR, a0R, a1R], axis=1)
    bid3 = bid.reshape(NB, 1, BN)

    x = _k0x(nif, nff, nitp, Wf, bf2)
    for l in range(NLAYER):
        Wsplit, b, t, was, wad = gat[l]
        haug, hs, hd = _kprep(x, Wsplit, was, wad)
        num, csum = _sc_edge_call(haug, idx4R,
                                  hs.reshape(N), hd.reshape(N),
                                  t.reshape(256), E)
        num = num[:, :N, :]
        csum = csum[0]
        if l < NLAYER - 1:
            x = _kmid(num, csum, haug, hs, hd, b, E)
        else:
            out = _kfin(num, csum, haug, hs, hd, b, bid3, E)
    return out


def kernel(aa_nodes_int_feats, aa_nodes_float_feats, aa_edge_attrs, aa_edges, aa_batch_id, mol_nodes_int_feats, mol_nodes_float_feats, mol_edge_attrs, mol_edges, mol_batch_id, neighbor_nodes_int_feats, neighbor_nodes_float_feats, neighbor_edge_attrs, neighbor_edges, neighbor_batch_id, nit_0, nit_1, nit_2, eet_0, eet_1, Wf, bf, gat0_W, gat0_b, gat0_att_src, gat0_att_dst, gat0_att_edge, gat0_We, gat1_W, gat1_b, gat1_att_src, gat1_att_dst, gat1_att_edge, gat1_We, gat2_W, gat2_b, gat2_att_src, gat2_att_dst, gat2_att_edge, gat2_We):
    pad28 = lambda m: jnp.pad(m, ((0, 28), (0, 0)))
    nitp = jnp.concatenate([pad28(nit_0), pad28(nit_1), pad28(nit_2)], axis=0)
    eets = jnp.concatenate([pad28(eet_0), pad28(eet_1)], axis=0)
    bf2 = bf.reshape(1, 128)

    gparams = [
        (gat0_W, gat0_b, gat0_att_src, gat0_att_dst, gat0_att_edge, gat0_We),
        (gat1_W, gat1_b, gat1_att_src, gat1_att_dst, gat1_att_edge, gat1_We),
        (gat2_W, gat2_b, gat2_att_src, gat2_att_dst, gat2_att_edge, gat2_We),
    ]
    gat = []
    for (W, b, a_s, a_d, a_e, We) in gparams:
        t, was, wad = _atttab(eets, We, a_e.reshape(128, 1), W,
                              a_s.reshape(128, 1), a_d.reshape(128, 1))
        Wsplit = jnp.stack([W[:, 0:CH], W[:, CH:2 * CH]])
        gat.append((Wsplit, b.reshape(1, 128), t, was, wad))

    graphs = [
        (aa_nodes_int_feats, aa_nodes_float_feats, aa_edge_attrs, aa_edges,
         aa_batch_id, 320000),
        (mol_nodes_int_feats, mol_nodes_float_feats, mol_edge_attrs,
         mol_edges, mol_batch_id, 160000),
        (neighbor_nodes_int_feats, neighbor_nodes_float_feats,
         neighbor_edge_attrs, neighbor_edges, neighbor_batch_id, 160000),
    ]
    outs = [_graph_forward(nif, nff, ea, ei, bid, nitp, Wf, bf2, gat, E)
            for (nif, nff, ea, ei, bid, E) in graphs]
    return jnp.concatenate(outs, axis=0)
